# baseline (speedup 1.0000x reference)
import jax
import jax.numpy as jnp
import numpy as np
from jax import lax
from jax.experimental import pallas as pl
from jax.experimental.pallas import tpu as pltpu
from jax.experimental.pallas import tpu_sc as plsc

CUTOFF = 0.05
BUCKETS_PER_CUTOFF = 1
EXTRA_SPACE = 1e-05

_static_bound = (np.ones(3, np.float32) * CUTOFF / BUCKETS_PER_CUTOFF
                 + EXTRA_SPACE).astype(np.float32)
_grid = np.floor(np.ones(3, np.float32) / _static_bound).astype(np.int32)
TOTAL_BUCKETS = int(np.prod(_grid))
SCALE0 = int(_grid[1]) * int(_grid[2])
SCALE1 = int(_grid[1])

N = 500000
W = 32
CHUNK = 15632
PADN = W * CHUNK
NVEC = CHUNK // 16
NVEC_LAST = (N - (W - 1) * CHUNK) // 16
NB = 8192
RNG = NB // W
SCAT = 15744
NCH = SCAT // 128
NPADV = (SCAT - CHUNK) // 16
NPADV_LAST = (SCAT - (NVEC_LAST * 16)) // 16

_mesh = plsc.VectorSubcoreMesh(core_axis_name="c", subcore_axis_name="s")
_params = pltpu.CompilerParams(needs_layout_passes=False)


def _wid():
    return lax.axis_index("s") * 2 + lax.axis_index("c")


def _k1_body(coords_hbm, cvec_hbm, flat_hbm, rank_hbm, hist_hbm,
             coords_v, cv, keyv, rankv, rcount):
    wid = _wid()
    pltpu.sync_copy(coords_hbm.at[pl.ds(wid * (CHUNK * 3), CHUNK * 3)],
                    coords_v)
    pltpu.sync_copy(cvec_hbm, cv)

    def zbody(i, _):
        rcount[pl.ds(i * 16, 16)] = jnp.zeros((16,), jnp.int32)
        return 0
    lax.fori_loop(0, NB // 16, zbody, 0)

    dx = cv[pl.ds(0, 16)]
    dy = cv[pl.ds(16, 16)]
    dz = cv[pl.ds(32, 16)]
    gx = cv[pl.ds(48, 16)]
    gy = cv[pl.ds(64, 16)]
    gz = cv[pl.ds(80, 16)]
    lane = lax.iota(jnp.int32, 16)
    lane3 = lane * 3
    one = jnp.float32(1.0)
    zero = jnp.float32(0.0)

    def axis_bucket(xc, dc, gc):
        f0 = xc / dc
        t = f0.astype(jnp.int32).astype(jnp.float32)
        fl = t - jnp.where(f0 < t, one, zero)
        fr = f0 - fl
        fr = jnp.where(fr >= one, fr - one, fr)
        fr = jnp.where(fr < zero, fr + one, fr)
        p = fr * gc
        tp = p.astype(jnp.int32).astype(jnp.float32)
        pfl = tp - jnp.where(p < tp, one, zero)
        return pfl.astype(jnp.int32)

    nv = jnp.where(wid == W - 1, NVEC_LAST, NVEC)

    def body(i, _):
        b = i * 16
        cb = b * 3
        x = plsc.load_gather(coords_v, [lane3 + cb])
        y = plsc.load_gather(coords_v, [lane3 + cb + 1])
        z = plsc.load_gather(coords_v, [lane3 + cb + 2])
        key = (axis_bucket(x, dx, gx) * SCALE0
               + axis_bucket(y, dy, gy) * SCALE1
               + axis_bucket(z, dz, gz))
        base = plsc.load_gather(rcount, [key])
        d, lm = plsc.scan_count(key)
        r1 = base + d
        plsc.store_scatter(rcount, [key], r1, mask=lm)
        keyv[pl.ds(b, 16)] = key
        rankv[pl.ds(b, 16)] = r1 - 1
        return 0
    lax.fori_loop(0, nv, body, 0)

    pltpu.sync_copy(keyv, flat_hbm.at[pl.ds(wid * CHUNK, CHUNK)])
    pltpu.sync_copy(rankv, rank_hbm.at[pl.ds(wid * CHUNK, CHUNK)])
    pltpu.sync_copy(rcount, hist_hbm.at[wid])


def _k2_body(hist_hbm, count_hbm, preoffs_hbm, totals_hbm, maxs_hbm,
             histv, countv, ecv, tv, mv):
    wid = _wid()
    off = wid * RNG
    for t in range(W):
        pltpu.sync_copy(hist_hbm.at[t, pl.ds(off, RNG)], histv.at[t])

    def jbody(j, _):
        jb = j * 16
        acc = jnp.zeros((16,), jnp.int32)
        for t in range(W):
            v = histv[t, pl.ds(jb, 16)]
            histv[t, pl.ds(jb, 16)] = acc
            acc = acc + v
        countv[pl.ds(jb, 16)] = acc
        return 0
    lax.fori_loop(0, RNG // 16, jbody, 0)

    def ebody(j, carry):
        jb = j * 16
        v = countv[pl.ds(jb, 16)]
        cs = plsc.cumsum(v)
        ecv[pl.ds(jb, 16)] = cs - v + carry
        return carry + jnp.sum(v)
    total = lax.fori_loop(0, RNG // 16, ebody, jnp.int32(0))

    def mbody(j, m):
        return jnp.maximum(m, countv[pl.ds(j * 16, 16)])
    m = lax.fori_loop(0, RNG // 16, mbody, jnp.zeros((16,), jnp.int32))

    tv[...] = jnp.full((16,), total, jnp.int32)
    mv[...] = jnp.full((16,), jnp.max(m), jnp.int32)

    def abody(j, _):
        jb = j * 16
        e = ecv[pl.ds(jb, 16)]
        for t in range(W):
            histv[t, pl.ds(jb, 16)] = histv[t, pl.ds(jb, 16)] + e
        return 0
    lax.fori_loop(0, RNG // 16, abody, 0)

    pltpu.sync_copy(countv, count_hbm.at[pl.ds(off, RNG)])
    for t in range(W):
        pltpu.sync_copy(histv.at[t], preoffs_hbm.at[t, pl.ds(off, RNG)])
    pltpu.sync_copy(tv, totals_hbm.at[wid])
    pltpu.sync_copy(mv, maxs_hbm.at[wid])


def _k3_body(flat_hbm, rank_hbm, preoffs_hbm, totals_hbm, maxs_hbm,
             cum_hbm, imidx_hbm, atidx_hbm, maxo_hbm,
             offsv, tvv, mvv, rbv, keyv, rankv, imv, idsv, mx16, sem):
    wid = _wid()
    pltpu.sync_copy(preoffs_hbm.at[wid], offsv)
    pltpu.sync_copy(totals_hbm, tvv)
    lane = lax.iota(jnp.int32, 16)
    zeros16 = jnp.zeros((16,), jnp.int32)

    v1 = plsc.load_gather(tvv, [lane, zeros16])
    v2 = plsc.load_gather(tvv, [lane + 16, zeros16])
    cs1 = plsc.cumsum(v1)
    ex1 = cs1 - v1
    s1 = jnp.sum(v1)
    cs2 = plsc.cumsum(v2)
    ex2 = cs2 - v2 + s1
    rbv[pl.ds(0, 16)] = ex1
    rbv[pl.ds(16, 16)] = ex2

    def obody(j, _):
        r = j >> 4
        rb = plsc.load_gather(rbv, [jnp.full((16,), r, jnp.int32)])
        offsv[pl.ds(j * 16, 16)] = offsv[pl.ds(j * 16, 16)] + rb
        return 0
    lax.fori_loop(0, NB // 16, obody, 0)

    @pl.when(wid == 0)
    def _():
        pltpu.sync_copy(offsv, cum_hbm)
        pltpu.sync_copy(maxs_hbm, mvv)
        m1 = plsc.load_gather(mvv, [lane, zeros16])
        m2 = plsc.load_gather(mvv, [lane + 16, zeros16])
        mx16[...] = jnp.full((16,), jnp.max(jnp.maximum(m1, m2)), jnp.int32)
        pltpu.sync_copy(mx16, maxo_hbm)

    pltpu.sync_copy(flat_hbm.at[pl.ds(wid * CHUNK, CHUNK)], keyv)
    pltpu.sync_copy(rank_hbm.at[pl.ds(wid * CHUNK, CHUNK)], rankv)

    nv = jnp.where(wid == W - 1, NVEC_LAST, NVEC)
    idbase = wid * CHUNK

    def body(i, _):
        b = i * 16
        key = keyv[pl.ds(b, 16)]
        r0 = rankv[pl.ds(b, 16)]
        pos = plsc.load_gather(offsv, [key]) + r0
        imv[pl.ds(b, 16)] = pos
        idsv[pl.ds(b, 16)] = lane + (idbase + b)
        return 0
    lax.fori_loop(0, nv, body, 0)

    valid = nv * 16
    npadv = jnp.where(wid == W - 1, NPADV_LAST, NPADV)

    def pbody(k, _):
        o = valid + k * 16
        imv[pl.ds(o, 16)] = imv[pl.ds(k * 16, 16)]
        idsv[pl.ds(o, 16)] = idsv[pl.ds(k * 16, 16)]
        return 0
    lax.fori_loop(0, npadv, pbody, 0)

    pltpu.sync_copy(imv.at[pl.ds(0, CHUNK)],
                    imidx_hbm.at[pl.ds(wid * CHUNK, CHUNK)])

    handles = []
    for j in range(NCH):
        h = pltpu.async_copy(idsv.at[pl.ds(j * 128, 128)],
                             atidx_hbm.at[imv.at[pl.ds(j * 128, 128)]],
                             sem)
        handles.append(h)
        if len(handles) == 8 or j == NCH - 1:
            for h2 in handles:
                h2.wait()
            handles = []


_k1 = pl.kernel(
    _k1_body,
    out_type=(
        jax.ShapeDtypeStruct((PADN,), jnp.int32),
        jax.ShapeDtypeStruct((PADN,), jnp.int32),
        jax.ShapeDtypeStruct((W, NB), jnp.int32),
    ),
    mesh=_mesh,
    compiler_params=_params,
    scratch_types=[
        pltpu.VMEM((CHUNK * 3,), jnp.float32),
        pltpu.VMEM((96,), jnp.float32),
        pltpu.VMEM((CHUNK,), jnp.int32),
        pltpu.VMEM((CHUNK,), jnp.int32),
        pltpu.VMEM((NB,), jnp.int32),
    ],
)

_k2 = pl.kernel(
    _k2_body,
    out_type=(
        jax.ShapeDtypeStruct((NB,), jnp.int32),
        jax.ShapeDtypeStruct((W, NB), jnp.int32),
        jax.ShapeDtypeStruct((W, 16), jnp.int32),
        jax.ShapeDtypeStruct((W, 16), jnp.int32),
    ),
    mesh=_mesh,
    compiler_params=_params,
    scratch_types=[
        pltpu.VMEM((W, RNG), jnp.int32),
        pltpu.VMEM((RNG,), jnp.int32),
        pltpu.VMEM((RNG,), jnp.int32),
        pltpu.VMEM((16,), jnp.int32),
        pltpu.VMEM((16,), jnp.int32),
    ],
)

_k3 = pl.kernel(
    _k3_body,
    out_type=(
        jax.ShapeDtypeStruct((NB,), jnp.int32),
        jax.ShapeDtypeStruct((PADN,), jnp.int32),
        jax.ShapeDtypeStruct((N,), jnp.int32),
        jax.ShapeDtypeStruct((16,), jnp.int32),
    ),
    mesh=_mesh,
    compiler_params=_params,
    scratch_types=[
        pltpu.VMEM((NB,), jnp.int32),
        pltpu.VMEM((W, 16), jnp.int32),
        pltpu.VMEM((W, 16), jnp.int32),
        pltpu.VMEM((32,), jnp.int32),
        pltpu.VMEM((CHUNK,), jnp.int32),
        pltpu.VMEM((CHUNK,), jnp.int32),
        pltpu.VMEM((SCAT,), jnp.int32),
        pltpu.VMEM((SCAT,), jnp.int32),
        pltpu.VMEM((16,), jnp.int32),
        pltpu.SemaphoreType.DMA,
    ],
)


def kernel(cell, coordinates):
    cell_diagonal = jnp.diagonal(cell)
    blb = (jnp.ones(3, jnp.float32) * CUTOFF / BUCKETS_PER_CUTOFF
           + EXTRA_SPACE)
    sbg = jnp.floor(
        cell_diagonal / blb.astype(cell_diagonal.dtype)).astype(jnp.int32)
    cvec = jnp.concatenate([
        jnp.repeat(cell_diagonal.astype(jnp.float32), 16),
        jnp.repeat(sbg.astype(jnp.float32), 16),
    ])
    coords_flat = jnp.pad(coordinates.reshape(-1), (0, (PADN - N) * 3))

    flat_p, rank_p, hist = _k1(coords_flat, cvec)
    count_p, preoffs, totals, maxs = _k2(hist)
    cum_p, imidx_p, atidx, maxo = _k3(flat_p, rank_p, preoffs, totals, maxs)

    return (flat_p[:N],
            count_p[:TOTAL_BUCKETS],
            cum_p[:TOTAL_BUCKETS],
            maxo[0],
            imidx_p[:N],
            atidx)


# speedup vs baseline: 1.0031x; 1.0031x over previous
import jax
import jax.numpy as jnp
import numpy as np
from jax import lax
from jax.experimental import pallas as pl
from jax.experimental.pallas import tpu as pltpu
from jax.experimental.pallas import tpu_sc as plsc

CUTOFF = 0.05
BUCKETS_PER_CUTOFF = 1
EXTRA_SPACE = 1e-05

_static_bound = (np.ones(3, np.float32) * CUTOFF / BUCKETS_PER_CUTOFF
                 + EXTRA_SPACE).astype(np.float32)
_grid = np.floor(np.ones(3, np.float32) / _static_bound).astype(np.int32)
TOTAL_BUCKETS = int(np.prod(_grid))
SCALE0 = int(_grid[1]) * int(_grid[2])
SCALE1 = int(_grid[1])

N = 500000
W = 32
CHUNK = 15632
PADN = W * CHUNK
NVEC = CHUNK // 16
NVEC_LAST = (N - (W - 1) * CHUNK) // 16
NB = 8192
RNG = NB // W
SCAT = 15744
NCH = SCAT // 128
NPADV = (SCAT - CHUNK) // 16
NPADV_LAST = (SCAT - (NVEC_LAST * 16)) // 16

_mesh = plsc.VectorSubcoreMesh(core_axis_name="c", subcore_axis_name="s")
_params = pltpu.CompilerParams(needs_layout_passes=False)


def _wid():
    return lax.axis_index("s") * 2 + lax.axis_index("c")


CHUNK_LAST = NVEC_LAST * 16


def _k1_body(coords_hbm, cvec_hbm, flat_hbm, rank_hbm, hist_hbm,
             coords_v, cv, keyv, rankv, rcount):
    wid = _wid()

    @pl.when(wid != W - 1)
    def _():
        pltpu.sync_copy(coords_hbm.at[pl.ds(wid * (CHUNK * 3), CHUNK * 3)],
                        coords_v)

    @pl.when(wid == W - 1)
    def _():
        pltpu.sync_copy(coords_hbm.at[pl.ds((W - 1) * (CHUNK * 3),
                                            CHUNK_LAST * 3)],
                        coords_v.at[pl.ds(0, CHUNK_LAST * 3)])

    pltpu.sync_copy(cvec_hbm, cv)

    def zbody(i, _):
        rcount[pl.ds(i * 16, 16)] = jnp.zeros((16,), jnp.int32)
        return 0
    lax.fori_loop(0, NB // 16, zbody, 0)

    dx = cv[pl.ds(0, 16)]
    dy = cv[pl.ds(16, 16)]
    dz = cv[pl.ds(32, 16)]
    gx = cv[pl.ds(48, 16)]
    gy = cv[pl.ds(64, 16)]
    gz = cv[pl.ds(80, 16)]
    lane = lax.iota(jnp.int32, 16)
    lane3 = lane * 3
    one = jnp.float32(1.0)
    zero = jnp.float32(0.0)

    def axis_bucket(xc, dc, gc):
        f0 = xc / dc
        t = f0.astype(jnp.int32).astype(jnp.float32)
        fl = t - jnp.where(f0 < t, one, zero)
        fr = f0 - fl
        fr = jnp.where(fr >= one, fr - one, fr)
        fr = jnp.where(fr < zero, fr + one, fr)
        p = fr * gc
        tp = p.astype(jnp.int32).astype(jnp.float32)
        pfl = tp - jnp.where(p < tp, one, zero)
        return pfl.astype(jnp.int32)

    nv = jnp.where(wid == W - 1, NVEC_LAST, NVEC)

    def body(i, _):
        b = i * 16
        cb = b * 3
        x = plsc.load_gather(coords_v, [lane3 + cb])
        y = plsc.load_gather(coords_v, [lane3 + cb + 1])
        z = plsc.load_gather(coords_v, [lane3 + cb + 2])
        key = (axis_bucket(x, dx, gx) * SCALE0
               + axis_bucket(y, dy, gy) * SCALE1
               + axis_bucket(z, dz, gz))
        base = plsc.load_gather(rcount, [key])
        d, lm = plsc.scan_count(key)
        r1 = base + d
        plsc.store_scatter(rcount, [key], r1, mask=lm)
        keyv[pl.ds(b, 16)] = key
        rankv[pl.ds(b, 16)] = r1 - 1
        return 0
    lax.fori_loop(0, nv, body, 0)

    @pl.when(wid != W - 1)
    def _():
        pltpu.sync_copy(keyv, flat_hbm.at[pl.ds(wid * CHUNK, CHUNK)])
        pltpu.sync_copy(rankv, rank_hbm.at[pl.ds(wid * CHUNK, CHUNK)])

    @pl.when(wid == W - 1)
    def _():
        pltpu.sync_copy(keyv.at[pl.ds(0, CHUNK_LAST)],
                        flat_hbm.at[pl.ds((W - 1) * CHUNK, CHUNK_LAST)])
        pltpu.sync_copy(rankv.at[pl.ds(0, CHUNK_LAST)],
                        rank_hbm.at[pl.ds((W - 1) * CHUNK, CHUNK_LAST)])

    pltpu.sync_copy(rcount, hist_hbm.at[wid])


def _k2_body(hist_hbm, count_hbm, preoffs_hbm, totals_hbm, maxs_hbm,
             histv, countv, ecv, tv, mv):
    wid = _wid()
    off = wid * RNG
    for t in range(W):
        pltpu.sync_copy(hist_hbm.at[t, pl.ds(off, RNG)], histv.at[t])

    def jbody(j, _):
        jb = j * 16
        acc = jnp.zeros((16,), jnp.int32)
        for t in range(W):
            v = histv[t, pl.ds(jb, 16)]
            histv[t, pl.ds(jb, 16)] = acc
            acc = acc + v
        countv[pl.ds(jb, 16)] = acc
        return 0
    lax.fori_loop(0, RNG // 16, jbody, 0)

    def ebody(j, carry):
        jb = j * 16
        v = countv[pl.ds(jb, 16)]
        cs = plsc.cumsum(v)
        ecv[pl.ds(jb, 16)] = cs - v + carry
        return carry + jnp.sum(v)
    total = lax.fori_loop(0, RNG // 16, ebody, jnp.int32(0))

    def mbody(j, m):
        return jnp.maximum(m, countv[pl.ds(j * 16, 16)])
    m = lax.fori_loop(0, RNG // 16, mbody, jnp.zeros((16,), jnp.int32))

    tv[...] = jnp.full((16,), total, jnp.int32)
    mv[...] = jnp.full((16,), jnp.max(m), jnp.int32)

    def abody(j, _):
        jb = j * 16
        e = ecv[pl.ds(jb, 16)]
        for t in range(W):
            histv[t, pl.ds(jb, 16)] = histv[t, pl.ds(jb, 16)] + e
        return 0
    lax.fori_loop(0, RNG // 16, abody, 0)

    FULL_R = TOTAL_BUCKETS // RNG
    TAIL = TOTAL_BUCKETS - FULL_R * RNG

    @pl.when(wid < FULL_R)
    def _():
        pltpu.sync_copy(countv, count_hbm.at[pl.ds(off, RNG)])

    @pl.when(wid == FULL_R)
    def _():
        pltpu.sync_copy(countv.at[pl.ds(0, TAIL)],
                        count_hbm.at[pl.ds(FULL_R * RNG, TAIL)])

    for t in range(W):
        pltpu.sync_copy(histv.at[t], preoffs_hbm.at[t, pl.ds(off, RNG)])
    pltpu.sync_copy(tv, totals_hbm.at[wid])
    pltpu.sync_copy(mv, maxs_hbm.at[wid])


def _k3_body(flat_hbm, rank_hbm, preoffs_hbm, totals_hbm, maxs_hbm,
             cum_hbm, imidx_hbm, atidx_hbm, maxo_hbm,
             offsv, tvv, mvv, rbv, keyv, rankv, imv, idsv, mx16, sem):
    wid = _wid()
    pltpu.sync_copy(preoffs_hbm.at[wid], offsv)
    pltpu.sync_copy(totals_hbm, tvv)
    lane = lax.iota(jnp.int32, 16)
    zeros16 = jnp.zeros((16,), jnp.int32)

    v1 = plsc.load_gather(tvv, [lane, zeros16])
    v2 = plsc.load_gather(tvv, [lane + 16, zeros16])
    cs1 = plsc.cumsum(v1)
    ex1 = cs1 - v1
    s1 = jnp.sum(v1)
    cs2 = plsc.cumsum(v2)
    ex2 = cs2 - v2 + s1
    rbv[pl.ds(0, 16)] = ex1
    rbv[pl.ds(16, 16)] = ex2

    def obody(j, _):
        r = j >> 4
        rb = plsc.load_gather(rbv, [jnp.full((16,), r, jnp.int32)])
        offsv[pl.ds(j * 16, 16)] = offsv[pl.ds(j * 16, 16)] + rb
        return 0
    lax.fori_loop(0, NB // 16, obody, 0)

    @pl.when(wid == 0)
    def _():
        pltpu.sync_copy(offsv.at[pl.ds(0, TOTAL_BUCKETS)], cum_hbm)
        pltpu.sync_copy(maxs_hbm, mvv)
        m1 = plsc.load_gather(mvv, [lane, zeros16])
        m2 = plsc.load_gather(mvv, [lane + 16, zeros16])
        mx16[...] = jnp.full((16,), jnp.max(jnp.maximum(m1, m2)), jnp.int32)
        pltpu.sync_copy(mx16, maxo_hbm)

    @pl.when(wid != W - 1)
    def _():
        pltpu.sync_copy(flat_hbm.at[pl.ds(wid * CHUNK, CHUNK)], keyv)
        pltpu.sync_copy(rank_hbm.at[pl.ds(wid * CHUNK, CHUNK)], rankv)

    @pl.when(wid == W - 1)
    def _():
        pltpu.sync_copy(flat_hbm.at[pl.ds((W - 1) * CHUNK, CHUNK_LAST)],
                        keyv.at[pl.ds(0, CHUNK_LAST)])
        pltpu.sync_copy(rank_hbm.at[pl.ds((W - 1) * CHUNK, CHUNK_LAST)],
                        rankv.at[pl.ds(0, CHUNK_LAST)])

    nv = jnp.where(wid == W - 1, NVEC_LAST, NVEC)
    idbase = wid * CHUNK

    def body(i, _):
        b = i * 16
        key = keyv[pl.ds(b, 16)]
        r0 = rankv[pl.ds(b, 16)]
        pos = plsc.load_gather(offsv, [key]) + r0
        imv[pl.ds(b, 16)] = pos
        idsv[pl.ds(b, 16)] = lane + (idbase + b)
        return 0
    lax.fori_loop(0, nv, body, 0)

    valid = nv * 16
    npadv = jnp.where(wid == W - 1, NPADV_LAST, NPADV)

    def pbody(k, _):
        o = valid + k * 16
        imv[pl.ds(o, 16)] = imv[pl.ds(k * 16, 16)]
        idsv[pl.ds(o, 16)] = idsv[pl.ds(k * 16, 16)]
        return 0
    lax.fori_loop(0, npadv, pbody, 0)

    @pl.when(wid != W - 1)
    def _():
        pltpu.sync_copy(imv.at[pl.ds(0, CHUNK)],
                        imidx_hbm.at[pl.ds(wid * CHUNK, CHUNK)])

    @pl.when(wid == W - 1)
    def _():
        pltpu.sync_copy(imv.at[pl.ds(0, CHUNK_LAST)],
                        imidx_hbm.at[pl.ds((W - 1) * CHUNK, CHUNK_LAST)])

    handles = []
    for j in range(NCH):
        h = pltpu.async_copy(idsv.at[pl.ds(j * 128, 128)],
                             atidx_hbm.at[imv.at[pl.ds(j * 128, 128)]],
                             sem)
        handles.append(h)
        if len(handles) == 8 or j == NCH - 1:
            for h2 in handles:
                h2.wait()
            handles = []


_k1 = pl.kernel(
    _k1_body,
    out_type=(
        jax.ShapeDtypeStruct((N,), jnp.int32),
        jax.ShapeDtypeStruct((N,), jnp.int32),
        jax.ShapeDtypeStruct((W, NB), jnp.int32),
    ),
    mesh=_mesh,
    compiler_params=_params,
    scratch_types=[
        pltpu.VMEM((CHUNK * 3,), jnp.float32),
        pltpu.VMEM((96,), jnp.float32),
        pltpu.VMEM((CHUNK,), jnp.int32),
        pltpu.VMEM((CHUNK,), jnp.int32),
        pltpu.VMEM((NB,), jnp.int32),
    ],
)

_k2 = pl.kernel(
    _k2_body,
    out_type=(
        jax.ShapeDtypeStruct((TOTAL_BUCKETS,), jnp.int32),
        jax.ShapeDtypeStruct((W, NB), jnp.int32),
        jax.ShapeDtypeStruct((W, 16), jnp.int32),
        jax.ShapeDtypeStruct((W, 16), jnp.int32),
    ),
    mesh=_mesh,
    compiler_params=_params,
    scratch_types=[
        pltpu.VMEM((W, RNG), jnp.int32),
        pltpu.VMEM((RNG,), jnp.int32),
        pltpu.VMEM((RNG,), jnp.int32),
        pltpu.VMEM((16,), jnp.int32),
        pltpu.VMEM((16,), jnp.int32),
    ],
)

_k3 = pl.kernel(
    _k3_body,
    out_type=(
        jax.ShapeDtypeStruct((TOTAL_BUCKETS,), jnp.int32),
        jax.ShapeDtypeStruct((N,), jnp.int32),
        jax.ShapeDtypeStruct((N,), jnp.int32),
        jax.ShapeDtypeStruct((16,), jnp.int32),
    ),
    mesh=_mesh,
    compiler_params=_params,
    scratch_types=[
        pltpu.VMEM((NB,), jnp.int32),
        pltpu.VMEM((W, 16), jnp.int32),
        pltpu.VMEM((W, 16), jnp.int32),
        pltpu.VMEM((32,), jnp.int32),
        pltpu.VMEM((CHUNK,), jnp.int32),
        pltpu.VMEM((CHUNK,), jnp.int32),
        pltpu.VMEM((SCAT,), jnp.int32),
        pltpu.VMEM((SCAT,), jnp.int32),
        pltpu.VMEM((16,), jnp.int32),
        pltpu.SemaphoreType.DMA,
    ],
)


def kernel(cell, coordinates):
    cell_diagonal = jnp.diagonal(cell)
    blb = (jnp.ones(3, jnp.float32) * CUTOFF / BUCKETS_PER_CUTOFF
           + EXTRA_SPACE)
    sbg = jnp.floor(
        cell_diagonal / blb.astype(cell_diagonal.dtype)).astype(jnp.int32)
    cvec = jnp.concatenate([
        jnp.repeat(cell_diagonal.astype(jnp.float32), 16),
        jnp.repeat(sbg.astype(jnp.float32), 16),
    ])
    coords_flat = coordinates.reshape(-1)

    flat_idx, rank_loc, hist = _k1(coords_flat, cvec)
    count, preoffs, totals, maxs = _k2(hist)
    cum, imidx, atidx, maxo = _k3(flat_idx, rank_loc, preoffs, totals, maxs)

    return (flat_idx, count, cum, maxo[0], imidx, atidx)


# speedup vs baseline: 3.4159x; 3.4053x over previous
import jax
import jax.numpy as jnp
import numpy as np
from jax import lax
from jax.experimental import pallas as pl
from jax.experimental.pallas import tpu as pltpu
from jax.experimental.pallas import tpu_sc as plsc

CUTOFF = 0.05
BUCKETS_PER_CUTOFF = 1
EXTRA_SPACE = 1e-05

_static_bound = (np.ones(3, np.float32) * CUTOFF / BUCKETS_PER_CUTOFF
                 + EXTRA_SPACE).astype(np.float32)
_grid = np.floor(np.ones(3, np.float32) / _static_bound).astype(np.int32)
TOTAL_BUCKETS = int(np.prod(_grid))
SCALE0 = int(_grid[1]) * int(_grid[2])
SCALE1 = int(_grid[1])

N = 500000
W = 32
CHUNK = 15632
PADN = W * CHUNK
NVEC = CHUNK // 16
NVEC_LAST = (N - (W - 1) * CHUNK) // 16
NB = 8192
RNG = NB // W
SCAT = 15744
NCH = SCAT // 128
NPADV = (SCAT - CHUNK) // 16
NPADV_LAST = (SCAT - (NVEC_LAST * 16)) // 16

_mesh = plsc.VectorSubcoreMesh(core_axis_name="c", subcore_axis_name="s")
_params = pltpu.CompilerParams(needs_layout_passes=False)


def _wid():
    return lax.axis_index("s") * 2 + lax.axis_index("c")


CHUNK_LAST = NVEC_LAST * 16


def _k1_body(xs_hbm, ys_hbm, zs_hbm, cvec_hbm, flat_hbm, rank_hbm, hist_hbm,
             xv, yv, zv, cv, keyv, rankv, rcount):
    wid = _wid()

    @pl.when(wid != W - 1)
    def _():
        pltpu.sync_copy(xs_hbm.at[pl.ds(wid * CHUNK, CHUNK)], xv)
        pltpu.sync_copy(ys_hbm.at[pl.ds(wid * CHUNK, CHUNK)], yv)
        pltpu.sync_copy(zs_hbm.at[pl.ds(wid * CHUNK, CHUNK)], zv)

    @pl.when(wid == W - 1)
    def _():
        pltpu.sync_copy(xs_hbm.at[pl.ds((W - 1) * CHUNK, CHUNK_LAST)],
                        xv.at[pl.ds(0, CHUNK_LAST)])
        pltpu.sync_copy(ys_hbm.at[pl.ds((W - 1) * CHUNK, CHUNK_LAST)],
                        yv.at[pl.ds(0, CHUNK_LAST)])
        pltpu.sync_copy(zs_hbm.at[pl.ds((W - 1) * CHUNK, CHUNK_LAST)],
                        zv.at[pl.ds(0, CHUNK_LAST)])

    pltpu.sync_copy(cvec_hbm, cv)

    def zbody(i, _):
        rcount[pl.ds(i * 16, 16)] = jnp.zeros((16,), jnp.int32)
        return 0
    lax.fori_loop(0, NB // 16, zbody, 0)

    dx = cv[pl.ds(0, 16)]
    dy = cv[pl.ds(16, 16)]
    dz = cv[pl.ds(32, 16)]
    gx = cv[pl.ds(48, 16)]
    gy = cv[pl.ds(64, 16)]
    gz = cv[pl.ds(80, 16)]
    one = jnp.float32(1.0)
    zero = jnp.float32(0.0)

    def axis_bucket(xc, dc, gc):
        f0 = xc / dc
        t = f0.astype(jnp.int32).astype(jnp.float32)
        fl = t - jnp.where(f0 < t, one, zero)
        fr = f0 - fl
        fr = jnp.where(fr >= one, fr - one, fr)
        fr = jnp.where(fr < zero, fr + one, fr)
        p = fr * gc
        tp = p.astype(jnp.int32).astype(jnp.float32)
        pfl = tp - jnp.where(p < tp, one, zero)
        return pfl.astype(jnp.int32)

    nv = jnp.where(wid == W - 1, NVEC_LAST, NVEC)

    def body(i, _):
        b = i * 16
        x = xv[pl.ds(b, 16)]
        y = yv[pl.ds(b, 16)]
        z = zv[pl.ds(b, 16)]
        key = (axis_bucket(x, dx, gx) * SCALE0
               + axis_bucket(y, dy, gy) * SCALE1
               + axis_bucket(z, dz, gz))
        base = plsc.load_gather(rcount, [key])
        d, lm = plsc.scan_count(key)
        r1 = base + d
        plsc.store_scatter(rcount, [key], r1, mask=lm)
        keyv[pl.ds(b, 16)] = key
        rankv[pl.ds(b, 16)] = r1 - 1
        return 0
    lax.fori_loop(0, nv, body, 0)

    @pl.when(wid != W - 1)
    def _():
        pltpu.sync_copy(keyv, flat_hbm.at[pl.ds(wid * CHUNK, CHUNK)])
        pltpu.sync_copy(rankv, rank_hbm.at[pl.ds(wid * CHUNK, CHUNK)])

    @pl.when(wid == W - 1)
    def _():
        pltpu.sync_copy(keyv.at[pl.ds(0, CHUNK_LAST)],
                        flat_hbm.at[pl.ds((W - 1) * CHUNK, CHUNK_LAST)])
        pltpu.sync_copy(rankv.at[pl.ds(0, CHUNK_LAST)],
                        rank_hbm.at[pl.ds((W - 1) * CHUNK, CHUNK_LAST)])

    pltpu.sync_copy(rcount, hist_hbm.at[wid])


def _k2_body(hist_hbm, count_hbm, preoffs_hbm, totals_hbm, maxs_hbm,
             histv, countv, ecv, tv, mv):
    wid = _wid()
    off = wid * RNG
    for t in range(W):
        pltpu.sync_copy(hist_hbm.at[t, pl.ds(off, RNG)], histv.at[t])

    def jbody(j, _):
        jb = j * 16
        acc = jnp.zeros((16,), jnp.int32)
        for t in range(W):
            v = histv[t, pl.ds(jb, 16)]
            histv[t, pl.ds(jb, 16)] = acc
            acc = acc + v
        countv[pl.ds(jb, 16)] = acc
        return 0
    lax.fori_loop(0, RNG // 16, jbody, 0)

    def ebody(j, carry):
        jb = j * 16
        v = countv[pl.ds(jb, 16)]
        cs = plsc.cumsum(v)
        ecv[pl.ds(jb, 16)] = cs - v + carry
        return carry + jnp.sum(v)
    total = lax.fori_loop(0, RNG // 16, ebody, jnp.int32(0))

    def mbody(j, m):
        return jnp.maximum(m, countv[pl.ds(j * 16, 16)])
    m = lax.fori_loop(0, RNG // 16, mbody, jnp.zeros((16,), jnp.int32))

    tv[...] = jnp.full((16,), total, jnp.int32)
    mv[...] = jnp.full((16,), jnp.max(m), jnp.int32)

    def abody(j, _):
        jb = j * 16
        e = ecv[pl.ds(jb, 16)]
        for t in range(W):
            histv[t, pl.ds(jb, 16)] = histv[t, pl.ds(jb, 16)] + e
        return 0
    lax.fori_loop(0, RNG // 16, abody, 0)

    FULL_R = TOTAL_BUCKETS // RNG
    TAIL = TOTAL_BUCKETS - FULL_R * RNG

    @pl.when(wid < FULL_R)
    def _():
        pltpu.sync_copy(countv, count_hbm.at[pl.ds(off, RNG)])

    @pl.when(wid == FULL_R)
    def _():
        pltpu.sync_copy(countv.at[pl.ds(0, TAIL)],
                        count_hbm.at[pl.ds(FULL_R * RNG, TAIL)])

    for t in range(W):
        pltpu.sync_copy(histv.at[t], preoffs_hbm.at[t, pl.ds(off, RNG)])
    pltpu.sync_copy(tv, totals_hbm.at[wid])
    pltpu.sync_copy(mv, maxs_hbm.at[wid])


def _k3_body(flat_hbm, rank_hbm, preoffs_hbm, totals_hbm, maxs_hbm,
             cum_hbm, imidx_hbm, atidx_hbm, maxo_hbm,
             offsv, tvv, mvv, rbv, keyv, rankv, imv, idsv, mx16, sem):
    wid = _wid()
    pltpu.sync_copy(preoffs_hbm.at[wid], offsv)
    pltpu.sync_copy(totals_hbm, tvv)
    lane = lax.iota(jnp.int32, 16)
    zeros16 = jnp.zeros((16,), jnp.int32)

    v1 = plsc.load_gather(tvv, [lane, zeros16])
    v2 = plsc.load_gather(tvv, [lane + 16, zeros16])
    cs1 = plsc.cumsum(v1)
    ex1 = cs1 - v1
    s1 = jnp.sum(v1)
    cs2 = plsc.cumsum(v2)
    ex2 = cs2 - v2 + s1
    rbv[pl.ds(0, 16)] = ex1
    rbv[pl.ds(16, 16)] = ex2

    def obody(j, _):
        r = j >> 4
        rb = plsc.load_gather(rbv, [jnp.full((16,), r, jnp.int32)])
        offsv[pl.ds(j * 16, 16)] = offsv[pl.ds(j * 16, 16)] + rb
        return 0
    lax.fori_loop(0, NB // 16, obody, 0)

    @pl.when(wid == 0)
    def _():
        pltpu.sync_copy(offsv.at[pl.ds(0, TOTAL_BUCKETS)], cum_hbm)
        pltpu.sync_copy(maxs_hbm, mvv)
        m1 = plsc.load_gather(mvv, [lane, zeros16])
        m2 = plsc.load_gather(mvv, [lane + 16, zeros16])
        mx16[...] = jnp.full((16,), jnp.max(jnp.maximum(m1, m2)), jnp.int32)
        pltpu.sync_copy(mx16, maxo_hbm)

    @pl.when(wid != W - 1)
    def _():
        pltpu.sync_copy(flat_hbm.at[pl.ds(wid * CHUNK, CHUNK)], keyv)
        pltpu.sync_copy(rank_hbm.at[pl.ds(wid * CHUNK, CHUNK)], rankv)

    @pl.when(wid == W - 1)
    def _():
        pltpu.sync_copy(flat_hbm.at[pl.ds((W - 1) * CHUNK, CHUNK_LAST)],
                        keyv.at[pl.ds(0, CHUNK_LAST)])
        pltpu.sync_copy(rank_hbm.at[pl.ds((W - 1) * CHUNK, CHUNK_LAST)],
                        rankv.at[pl.ds(0, CHUNK_LAST)])

    nv = jnp.where(wid == W - 1, NVEC_LAST, NVEC)
    idbase = wid * CHUNK

    def body(i, _):
        b = i * 16
        key = keyv[pl.ds(b, 16)]
        r0 = rankv[pl.ds(b, 16)]
        pos = plsc.load_gather(offsv, [key]) + r0
        imv[pl.ds(b, 16)] = pos
        idsv[pl.ds(b, 16)] = lane + (idbase + b)
        return 0
    lax.fori_loop(0, nv, body, 0)

    valid = nv * 16
    npadv = jnp.where(wid == W - 1, NPADV_LAST, NPADV)

    def pbody(k, _):
        o = valid + k * 16
        imv[pl.ds(o, 16)] = imv[pl.ds(k * 16, 16)]
        idsv[pl.ds(o, 16)] = idsv[pl.ds(k * 16, 16)]
        return 0
    lax.fori_loop(0, npadv, pbody, 0)

    @pl.when(wid != W - 1)
    def _():
        pltpu.sync_copy(imv.at[pl.ds(0, CHUNK)],
                        imidx_hbm.at[pl.ds(wid * CHUNK, CHUNK)])

    @pl.when(wid == W - 1)
    def _():
        pltpu.sync_copy(imv.at[pl.ds(0, CHUNK_LAST)],
                        imidx_hbm.at[pl.ds((W - 1) * CHUNK, CHUNK_LAST)])

    handles = []
    for j in range(NCH):
        h = pltpu.async_copy(idsv.at[pl.ds(j * 128, 128)],
                             atidx_hbm.at[imv.at[pl.ds(j * 128, 128)]],
                             sem)
        handles.append(h)
        if len(handles) == 8 or j == NCH - 1:
            for h2 in handles:
                h2.wait()
            handles = []


_k1 = pl.kernel(
    _k1_body,
    out_type=(
        jax.ShapeDtypeStruct((N,), jnp.int32),
        jax.ShapeDtypeStruct((N,), jnp.int32),
        jax.ShapeDtypeStruct((W, NB), jnp.int32),
    ),
    mesh=_mesh,
    compiler_params=_params,
    scratch_types=[
        pltpu.VMEM((CHUNK,), jnp.float32),
        pltpu.VMEM((CHUNK,), jnp.float32),
        pltpu.VMEM((CHUNK,), jnp.float32),
        pltpu.VMEM((96,), jnp.float32),
        pltpu.VMEM((CHUNK,), jnp.int32),
        pltpu.VMEM((CHUNK,), jnp.int32),
        pltpu.VMEM((NB,), jnp.int32),
    ],
)

_k2 = pl.kernel(
    _k2_body,
    out_type=(
        jax.ShapeDtypeStruct((TOTAL_BUCKETS,), jnp.int32),
        jax.ShapeDtypeStruct((W, NB), jnp.int32),
        jax.ShapeDtypeStruct((W, 16), jnp.int32),
        jax.ShapeDtypeStruct((W, 16), jnp.int32),
    ),
    mesh=_mesh,
    compiler_params=_params,
    scratch_types=[
        pltpu.VMEM((W, RNG), jnp.int32),
        pltpu.VMEM((RNG,), jnp.int32),
        pltpu.VMEM((RNG,), jnp.int32),
        pltpu.VMEM((16,), jnp.int32),
        pltpu.VMEM((16,), jnp.int32),
    ],
)

_k3 = pl.kernel(
    _k3_body,
    out_type=(
        jax.ShapeDtypeStruct((TOTAL_BUCKETS,), jnp.int32),
        jax.ShapeDtypeStruct((N,), jnp.int32),
        jax.ShapeDtypeStruct((N,), jnp.int32),
        jax.ShapeDtypeStruct((16,), jnp.int32),
    ),
    mesh=_mesh,
    compiler_params=_params,
    scratch_types=[
        pltpu.VMEM((NB,), jnp.int32),
        pltpu.VMEM((W, 16), jnp.int32),
        pltpu.VMEM((W, 16), jnp.int32),
        pltpu.VMEM((32,), jnp.int32),
        pltpu.VMEM((CHUNK,), jnp.int32),
        pltpu.VMEM((CHUNK,), jnp.int32),
        pltpu.VMEM((SCAT,), jnp.int32),
        pltpu.VMEM((SCAT,), jnp.int32),
        pltpu.VMEM((16,), jnp.int32),
        pltpu.SemaphoreType.DMA,
    ],
)


def kernel(cell, coordinates):
    cell_diagonal = jnp.diagonal(cell)
    blb = (jnp.ones(3, jnp.float32) * CUTOFF / BUCKETS_PER_CUTOFF
           + EXTRA_SPACE)
    sbg = jnp.floor(
        cell_diagonal / blb.astype(cell_diagonal.dtype)).astype(jnp.int32)
    cvec = jnp.concatenate([
        jnp.repeat(cell_diagonal.astype(jnp.float32), 16),
        jnp.repeat(sbg.astype(jnp.float32), 16),
    ])
    xs = coordinates[0, :, 0]
    ys = coordinates[0, :, 1]
    zs = coordinates[0, :, 2]

    flat_idx, rank_loc, hist = _k1(xs, ys, zs, cvec)
    count, preoffs, totals, maxs = _k2(hist)
    cum, imidx, atidx, maxo = _k3(flat_idx, rank_loc, preoffs, totals, maxs)

    return (flat_idx, count, cum, maxo[0], imidx, atidx)
